# baseline (speedup 1.0000x reference)
import functools

import jax
import jax.numpy as jnp
from jax import lax
from jax.experimental import pallas as pl
from jax.experimental.pallas import tpu as pltpu
from jax.experimental.pallas import tpu_sc as plsc

_DIM = 1024
_SEQ = 8192
_SCALE = _DIM ** (-0.5)
_NC = 2
_NS = 16
_NW = _NC * _NS
_RPW = _SEQ // _NW
_CHUNK = 32
_NCHUNKS = _RPW // _CHUNK
_NBUF = 3
_LOOK = 2
_LANES = 16

_mesh = plsc.VectorSubcoreMesh(core_axis_name="c", subcore_axis_name="s")


@functools.partial(
    pl.kernel,
    out_type=jax.ShapeDtypeStruct((_SEQ, _DIM), jnp.float32),
    mesh=_mesh,
    scratch_types=[
        pltpu.VMEM((_NBUF, _CHUNK, _DIM), jnp.float32),
        [pltpu.SemaphoreType.DMA] * _NBUF,
        [pltpu.SemaphoreType.DMA] * _NBUF,
    ],
)
def _sc_scale(emb_hbm, out_hbm, buf, sin, sout):
    wid = lax.axis_index("s") * _NC + lax.axis_index("c")
    base = wid * _RPW

    def start_in(c):
        b = c % _NBUF
        return pltpu.async_copy(
            emb_hbm.at[pl.ds(base + c * _CHUNK, _CHUNK)], buf.at[b], sin[b])

    def start_out(c):
        b = c % _NBUF
        return pltpu.async_copy(
            buf.at[b], out_hbm.at[pl.ds(base + c * _CHUNK, _CHUNK)], sout[b])

    def scale_buf(b):
        def body(i, carry):
            for u in range(4):
                r = i * 4 + u
                for j in range(_DIM // _LANES):
                    idx = (b, r, pl.ds(j * _LANES, _LANES))
                    buf[idx] = buf[idx] * _SCALE
            return carry
        lax.fori_loop(0, _CHUNK // 4, body, 0, unroll=False)

    d_in = {c: start_in(c) for c in range(min(_LOOK, _NCHUNKS))}
    d_out = {}
    out_pending = []
    for c in range(_NCHUNKS):
        d_in[c].wait()
        scale_buf(c % _NBUF)
        if c + _LOOK < _NCHUNKS:
            prev = c + _LOOK - _NBUF
            if prev >= 0:
                d_out[prev].wait()
                out_pending.remove(prev)
            d_in[c + _LOOK] = start_in(c + _LOOK)
        d_out[c] = start_out(c)
        out_pending.append(c)
    for c in out_pending:
        d_out[c].wait()


def kernel(x, emb):
    seq_len = x.shape[1]
    return _sc_scale(emb[:seq_len])


# speedup vs baseline: 1.3752x; 1.3752x over previous
import functools

import jax
import jax.numpy as jnp
from jax import lax
from jax.experimental import pallas as pl
from jax.experimental.pallas import tpu as pltpu
from jax.experimental.pallas import tpu_sc as plsc

_DIM = 1024
_SEQ = 8192
_SCALE = _DIM ** (-0.5)
_NC = 2
_NS = 16
_NW = _NC * _NS
_RPW = _SEQ // _NW
_CHUNK = 32
_NCHUNKS = _RPW // _CHUNK
_NBUF = 2
_NGROUPS = _NCHUNKS // _NBUF
_LANES = 16

_mesh = plsc.VectorSubcoreMesh(core_axis_name="c", subcore_axis_name="s")


@functools.partial(
    pl.kernel,
    out_type=jax.ShapeDtypeStruct((_SEQ, _DIM), jnp.float32),
    mesh=_mesh,
    scratch_types=[
        pltpu.VMEM((_NBUF, _CHUNK, _DIM), jnp.float32),
        [pltpu.SemaphoreType.DMA] * _NBUF,
        [pltpu.SemaphoreType.DMA] * _NBUF,
    ],
)
def _sc_scale(emb_hbm, out_hbm, buf, sin, sout):
    wid = lax.axis_index("s") * _NC + lax.axis_index("c")
    base = wid * _RPW

    def in_desc(c, b):
        return pltpu.make_async_copy(
            emb_hbm.at[pl.ds(base + c * _CHUNK, _CHUNK)], buf.at[b], sin[b])

    def out_desc(c, b):
        return pltpu.make_async_copy(
            buf.at[b], out_hbm.at[pl.ds(base + c * _CHUNK, _CHUNK)], sout[b])

    def scale_buf(b):
        def body(r, carry):
            for j in range(_DIM // _LANES):
                idx = (b, r, pl.ds(j * _LANES, _LANES))
                buf[idx] = buf[idx] * _SCALE
            return carry
        lax.fori_loop(0, _CHUNK, body, 0, unroll=False)

    in_desc(0, 0).start()
    in_desc(1, 1).start()

    def group(g, carry):
        for b in range(_NBUF):
            c = g * _NBUF + b
            in_desc(c, b).wait()
            scale_buf(b)

            @pl.when(g >= 1)
            def _wait_prev_out():
                out_desc(c - _NBUF, b).wait()

            @pl.when(c + _NBUF < _NCHUNKS)
            def _refill():
                in_desc(c + _NBUF, b).start()

            out_desc(c, b).start()
        return carry

    lax.fori_loop(0, _NGROUPS, group, 0, unroll=False)
    out_desc(_NCHUNKS - _NBUF, 0).wait()
    out_desc(_NCHUNKS - 1, 1).wait()


def kernel(x, emb):
    seq_len = x.shape[1]
    return _sc_scale(emb[:seq_len])
